# baseline (speedup 1.0000x reference)
import functools

import jax
import jax.numpy as jnp
import numpy as np
from jax import lax
from jax.experimental import pallas as pl
from jax.experimental.pallas import tpu as pltpu
from jax.experimental.pallas import tpu_sc as plsc

NOTES_POOL_SIZE = 1000000
EMBED_DIM = 32
B = 4096
L = 200

_NC = 2
_NS = 16
_NW = _NC * _NS
_BW = B // _NW
_CL = 4
_NCH = L // _CL
_CROWS = _CL * _BW
_TH = EMBED_DIM // 8

_TBLK = 4096
_TSUB = _TBLK // 4
_TGRID = -(-NOTES_POOL_SIZE // _TBLK)
_TROWS = _TGRID * _TBLK


def _positional_encoding_np(max_pos, embed_dim):
    pos = np.arange(max_pos)[:, np.newaxis]
    i = np.arange(embed_dim)[np.newaxis, :]
    angle_rates = 1.0 / np.power(10000, 2 * (i // 2) / np.float32(embed_dim))
    angle_rads = pos * angle_rates
    angle_rads[:, 0::2] = np.sin(angle_rads[:, 0::2])
    angle_rads[:, 1::2] = np.cos(angle_rads[:, 1::2])
    return angle_rads.astype(np.float32)


_POS_ENC = _positional_encoding_np(L, EMBED_DIM)


def _tc_table_shuffle(table_t):
    def body(t_ref, o_ref):
        blk = t_ref[...]
        ii = lax.broadcasted_iota(jnp.int32, (EMBED_DIM, 128), 0)
        jj = lax.broadcasted_iota(jnp.int32, (EMBED_DIM, 128), 1)
        acc = None
        for a in range(4):
            eye_a = (jj == ii + a * EMBED_DIM).astype(jnp.float32)
            p = lax.dot_general(
                blk[:, a * _TSUB:(a + 1) * _TSUB], eye_a,
                (((0,), (0,)), ((), ())),
                preferred_element_type=jnp.float32)
            acc = p if acc is None else acc + p
        o_ref[...] = acc

    return pl.pallas_call(
        body,
        grid=(_TGRID,),
        compiler_params=pltpu.CompilerParams(
            fuse_transposed_lhs_in_matmul=True),
        in_specs=[pl.BlockSpec((EMBED_DIM, _TBLK), lambda i: (0, i))],
        out_specs=pl.BlockSpec((_TSUB, 128), lambda i: (i, 0)),
        out_shape=jax.ShapeDtypeStruct((_TGRID * _TSUB, 128), jnp.float32),
    )(table_t)


def _sc_embed(tview, xv, pos):
    mesh = plsc.VectorSubcoreMesh(core_axis_name="c", subcore_axis_name="s")

    @functools.partial(
        pl.kernel,
        mesh=mesh,
        compiler_params=pltpu.CompilerParams(
            use_tc_tiling_on_sc=False, needs_layout_passes=False),
        out_type=jax.ShapeDtypeStruct((L, _TH, _NW, 8, _BW), jnp.float32),
        scratch_types=[
            pltpu.VMEM((2, _CL, 2, _BW), jnp.int32),
            pltpu.VMEM((2 * _CL, _BW), jnp.int32),
            pltpu.VMEM((2 * _CL, _BW), jnp.float32),
            pltpu.VMEM((2 * _CROWS, EMBED_DIM), jnp.float32),
            pltpu.VMEM((2, _CL, _TH, 8, _BW), jnp.float32),
            pltpu.VMEM((L * EMBED_DIM,), jnp.float32),
            pltpu.SemaphoreType.DMA,
            pltpu.SemaphoreType.DMA,
            pltpu.SemaphoreType.DMA,
            pltpu.SemaphoreType.DMA,
            pltpu.SemaphoreType.DMA,
            pltpu.SemaphoreType.DMA,
        ],
    )
    def k(tview_hbm, xv_hbm, pos_hbm, out_hbm,
          xin_v, idx_v, dur_v, rows_v, out_v, pos_v,
          lsem0, lsem1, gsem0, gsem1, osem0, osem1):
        wid = lax.axis_index("s") * _NC + lax.axis_index("c")
        iota16 = lax.iota(jnp.int32, 16)

        pltpu.sync_copy(pos_hbm, pos_v)

        def fire_loads(c, buf):
            lsem = [lsem0, lsem1][buf]
            for j in range(_CL):
                pltpu.async_copy(
                    xv_hbm.at[c * _CL + j, wid], xin_v.at[buf, j], lsem)

        def wait_loads(buf):
            lsem = [lsem0, lsem1][buf]
            for j in range(_CL):
                pltpu.make_async_copy(
                    xv_hbm.at[0, wid], xin_v.at[buf, j], lsem).wait()

        def prep(buf):
            for j in range(_CL):
                for g in range(_BW // 16):
                    sl = pl.ds(g * 16, 16)
                    n = xin_v[buf, j, 0, sl]
                    idx_v[buf * _CL + j, sl] = (
                        ((n >> 12) << 12) + ((n & (_TSUB - 1)) << 2)
                        + ((n >> 10) & 3))
                    dur_v[buf * _CL + j, sl] = (
                        xin_v[buf, j, 1, sl].astype(jnp.float32))

        def fire_gathers(buf):
            gsem = [gsem0, gsem1][buf]
            for j in range(_CL):
                pltpu.async_copy(
                    tview_hbm.at[idx_v.at[buf * _CL + j]],
                    rows_v.at[pl.ds(buf * _CROWS + j * _BW, _BW), :], gsem)

        def wait_gathers(buf):
            gsem = [gsem0, gsem1][buf]
            for j in range(_CL):
                pltpu.make_async_copy(
                    tview_hbm.at[idx_v.at[buf * _CL + j]],
                    rows_v.at[pl.ds(buf * _CROWS + j * _BW, _BW), :],
                    gsem).wait()

        def fire_outs(c, buf):
            osem = [osem0, osem1][buf]
            for dl in range(_CL):
                for th in range(_TH):
                    pltpu.async_copy(
                        out_v.at[buf, dl, th],
                        out_hbm.at[c * _CL + dl, th, wid], osem)

        def wait_outs(buf):
            osem = [osem0, osem1][buf]
            for dl in range(_CL):
                for th in range(_TH):
                    pltpu.make_async_copy(
                        out_v.at[buf, dl, th],
                        out_hbm.at[0, th, wid], osem).wait()

        th0 = iota16 >> 3
        dr0 = iota16 & 7
        th1 = th0 + 2

        def compute(c, buf):
            rbase = buf * _CROWS
            for dl in range(_CL):
                lpos = c * _CL + dl
                posh0 = pos_v[pl.ds(lpos * EMBED_DIM, 16)]
                posh1 = pos_v[pl.ds(lpos * EMBED_DIM + 16, 16)]
                i0 = jnp.full((16,), buf, jnp.int32)
                i1 = jnp.full((16,), dl, jnp.int32)

                @plsc.parallel_loop(0, _BW, unroll=8)
                def _sc_body(r, dl=dl, posh0=posh0, posh1=posh1,
                             i0=i0, i1=i1):
                    rr = rbase + dl * _BW + r
                    col = jnp.full((16,), r, jnp.int32)
                    dsp = plsc.load_gather(
                        dur_v, [jnp.full((16,), buf * _CL + dl, jnp.int32),
                                col])
                    plsc.store_scatter(
                        out_v, [i0, i1, th0, dr0, col],
                        rows_v[rr, pl.ds(0, 16)] + posh0 + dsp)
                    plsc.store_scatter(
                        out_v, [i0, i1, th1, dr0, col],
                        rows_v[rr, pl.ds(16, 16)] + posh1 + dsp)

        fire_loads(0, 0)
        fire_loads(1, 1)
        wait_loads(0)
        prep(0)
        fire_gathers(0)

        def step(t, _):
            for buf in range(2):
                c = 2 * t + buf
                nbuf = 1 - buf

                @pl.when(c >= 2)
                def _w():
                    wait_outs(buf)

                wait_gathers(buf)

                @pl.when(c + 1 < _NCH)
                def _g():
                    wait_loads(nbuf)
                    prep(nbuf)
                    fire_gathers(nbuf)

                compute(c, buf)
                fire_outs(c, buf)

                @pl.when(c + 2 < _NCH)
                def _l():
                    fire_loads(c + 2, buf)
            return _

        lax.fori_loop(0, _NCH // 2, step, 0)
        wait_outs(0)
        wait_outs(1)

    return k(tview, xv, pos)


@jax.jit
def kernel(x_in, table):
    trm = _tc_table_shuffle(table.T)
    tview = trm.reshape(_TROWS, EMBED_DIM)
    xv = x_in.reshape(_NW, _BW, L, 2).transpose(2, 0, 3, 1)
    pos = jnp.asarray(_POS_ENC).reshape(-1)
    out5 = _sc_embed(tview, xv, pos)
    return out5.transpose(2, 4, 0, 1, 3).reshape(B, L, EMBED_DIM)


# speedup vs baseline: 1.0146x; 1.0146x over previous
import functools

import jax
import jax.numpy as jnp
import numpy as np
from jax import lax
from jax.experimental import pallas as pl
from jax.experimental.pallas import tpu as pltpu
from jax.experimental.pallas import tpu_sc as plsc

NOTES_POOL_SIZE = 1000000
EMBED_DIM = 32
B = 4096
L = 200

_NC = 2
_NS = 16
_NW = _NC * _NS
_BW = B // _NW
_CL = 4
_NCH = L // _CL
_CROWS = _CL * _BW
_TH = EMBED_DIM // 8

_TBLK = 4096
_TSUB = _TBLK // 4
_TGRID = -(-NOTES_POOL_SIZE // _TBLK)
_TROWS = _TGRID * _TBLK


def _positional_encoding_np(max_pos, embed_dim):
    pos = np.arange(max_pos)[:, np.newaxis]
    i = np.arange(embed_dim)[np.newaxis, :]
    angle_rates = 1.0 / np.power(10000, 2 * (i // 2) / np.float32(embed_dim))
    angle_rads = pos * angle_rates
    angle_rads[:, 0::2] = np.sin(angle_rads[:, 0::2])
    angle_rads[:, 1::2] = np.cos(angle_rads[:, 1::2])
    return angle_rads.astype(np.float32)


_POS_ENC = _positional_encoding_np(L, EMBED_DIM)


def _tc_table_shuffle(table_t):
    def body(t_ref, o_ref):
        blk = t_ref[...]
        ii = lax.broadcasted_iota(jnp.int32, (EMBED_DIM, 128), 0)
        jj = lax.broadcasted_iota(jnp.int32, (EMBED_DIM, 128), 1)
        acc = None
        for a in range(4):
            eye_a = (jj == ii + a * EMBED_DIM).astype(jnp.float32)
            p = lax.dot_general(
                blk[:, a * _TSUB:(a + 1) * _TSUB], eye_a,
                (((0,), (0,)), ((), ())),
                preferred_element_type=jnp.float32)
            acc = p if acc is None else acc + p
        o_ref[...] = acc

    return pl.pallas_call(
        body,
        grid=(_TGRID,),
        compiler_params=pltpu.CompilerParams(
            fuse_transposed_lhs_in_matmul=True),
        in_specs=[pl.BlockSpec((EMBED_DIM, _TBLK), lambda i: (0, i))],
        out_specs=pl.BlockSpec((_TSUB, 128), lambda i: (i, 0)),
        out_shape=jax.ShapeDtypeStruct((_TGRID * _TSUB, 128), jnp.float32),
    )(table_t)


def _sc_embed(tview, xv, pos):
    mesh = plsc.VectorSubcoreMesh(core_axis_name="c", subcore_axis_name="s")

    @functools.partial(
        pl.kernel,
        mesh=mesh,
        compiler_params=pltpu.CompilerParams(
            use_tc_tiling_on_sc=False, needs_layout_passes=False),
        out_type=jax.ShapeDtypeStruct((L, _TH, _NW, 8, _BW), jnp.float32),
        scratch_types=[
            pltpu.VMEM((2, _CL, 2, _BW), jnp.int32),
            pltpu.VMEM((2 * _CL, _BW), jnp.int32),
            pltpu.VMEM((2 * _CL, _BW), jnp.float32),
            pltpu.VMEM((2 * _CROWS, EMBED_DIM), jnp.float32),
            pltpu.VMEM((2 * _CL * EMBED_DIM, _BW), jnp.float32),
            pltpu.VMEM((L * EMBED_DIM,), jnp.float32),
            pltpu.SemaphoreType.DMA,
            pltpu.SemaphoreType.DMA,
            pltpu.SemaphoreType.DMA,
            pltpu.SemaphoreType.DMA,
            pltpu.SemaphoreType.DMA,
            pltpu.SemaphoreType.DMA,
        ],
    )
    def k(tview_hbm, xv_hbm, pos_hbm, out_hbm,
          xin_v, idx_v, dur_v, rows_v, out_v, pos_v,
          lsem0, lsem1, gsem0, gsem1, osem0, osem1):
        wid = lax.axis_index("s") * _NC + lax.axis_index("c")
        iota16 = lax.iota(jnp.int32, 16)

        pltpu.sync_copy(pos_hbm, pos_v)

        def fire_loads(c, buf):
            lsem = [lsem0, lsem1][buf]
            for j in range(_CL):
                pltpu.async_copy(
                    xv_hbm.at[c * _CL + j, wid], xin_v.at[buf, j], lsem)

        def wait_loads(buf):
            lsem = [lsem0, lsem1][buf]
            for j in range(_CL):
                pltpu.make_async_copy(
                    xv_hbm.at[0, wid], xin_v.at[buf, j], lsem).wait()

        def prep(buf):
            for j in range(_CL):
                for g in range(_BW // 16):
                    sl = pl.ds(g * 16, 16)
                    n = xin_v[buf, j, 0, sl]
                    idx_v[buf * _CL + j, sl] = (
                        ((n >> 12) << 12) + ((n & (_TSUB - 1)) << 2)
                        + ((n >> 10) & 3))
                    dur_v[buf * _CL + j, sl] = (
                        xin_v[buf, j, 1, sl].astype(jnp.float32))

        def fire_gathers(buf):
            gsem = [gsem0, gsem1][buf]
            for j in range(_CL):
                pltpu.async_copy(
                    tview_hbm.at[idx_v.at[buf * _CL + j]],
                    rows_v.at[pl.ds(buf * _CROWS + j * _BW, _BW), :], gsem)

        def wait_gathers(buf):
            gsem = [gsem0, gsem1][buf]
            for j in range(_CL):
                pltpu.make_async_copy(
                    tview_hbm.at[idx_v.at[buf * _CL + j]],
                    rows_v.at[pl.ds(buf * _CROWS + j * _BW, _BW), :],
                    gsem).wait()

        def fire_outs(c, buf):
            osem = [osem0, osem1][buf]
            for dl in range(_CL):
                odl = (buf * _CL + dl) * EMBED_DIM
                for th in range(_TH):
                    pltpu.async_copy(
                        out_v.at[pl.ds(odl + th * 8, 8), :],
                        out_hbm.at[c * _CL + dl, th, wid], osem)

        def wait_outs(buf):
            osem = [osem0, osem1][buf]
            for dl in range(_CL):
                odl = (buf * _CL + dl) * EMBED_DIM
                for th in range(_TH):
                    pltpu.make_async_copy(
                        out_v.at[pl.ds(odl + th * 8, 8), :],
                        out_hbm.at[0, th, wid], osem).wait()

        def compute(c, buf):
            rbase = buf * _CROWS
            for dl in range(_CL):
                lpos = c * _CL + dl
                odl = (buf * _CL + dl) * EMBED_DIM
                posh0 = pos_v[pl.ds(lpos * EMBED_DIM, 16)]
                posh1 = pos_v[pl.ds(lpos * EMBED_DIM + 16, 16)]
                row0 = odl + iota16
                row1 = odl + 16 + iota16

                @plsc.parallel_loop(0, _BW, unroll=4)
                def _sc_body(r, dl=dl, posh0=posh0, posh1=posh1,
                             row0=row0, row1=row1):
                    rr = rbase + dl * _BW + r
                    col = jnp.full((16,), r, jnp.int32)
                    dsp = plsc.load_gather(
                        dur_v, [jnp.full((16,), buf * _CL + dl, jnp.int32),
                                col])
                    plsc.store_scatter(
                        out_v, [row0, col],
                        rows_v[rr, pl.ds(0, 16)] + posh0 + dsp)
                    plsc.store_scatter(
                        out_v, [row1, col],
                        rows_v[rr, pl.ds(16, 16)] + posh1 + dsp)

        fire_loads(0, 0)
        fire_loads(1, 1)
        wait_loads(0)
        prep(0)
        fire_gathers(0)

        def step(t, _):
            for buf in range(2):
                c = 2 * t + buf
                nbuf = 1 - buf

                @pl.when(c >= 2)
                def _w():
                    wait_outs(buf)

                wait_gathers(buf)

                @pl.when(c + 1 < _NCH)
                def _g():
                    wait_loads(nbuf)
                    prep(nbuf)
                    fire_gathers(nbuf)

                compute(c, buf)
                fire_outs(c, buf)

                @pl.when(c + 2 < _NCH)
                def _l():
                    fire_loads(c + 2, buf)
            return _

        lax.fori_loop(0, _NCH // 2, step, 0)
        wait_outs(0)
        wait_outs(1)

    return k(tview, xv, pos)


@jax.jit
def kernel(x_in, table):
    trm = _tc_table_shuffle(table.T)
    tview = trm.reshape(_TROWS, EMBED_DIM)
    xv = x_in.reshape(_NW, _BW, L, 2).transpose(2, 0, 3, 1)
    pos = jnp.asarray(_POS_ENC).reshape(-1)
    out5 = _sc_embed(tview, xv, pos)
    return out5.transpose(2, 4, 0, 1, 3).reshape(B, L, EMBED_DIM)
